# baseline (speedup 1.0000x reference)
import functools

import jax
import jax.numpy as jnp
from jax import lax
from jax.experimental import pallas as pl
from jax.experimental.pallas import tpu as pltpu
from jax.experimental.pallas import tpu_sc as plsc

_B, _L, _C = 2, 2048, 1024
_H, _K = 16, 64
_S = 33
_HALF = 16
_MAXF, _MINF = 16.0, 1.0
_MAXR = _HALF * _MAXF
_BL = _B * _L
_T = 256
_NBLK = _BL // _T

_NW = 32
_RPW = _BL // _NW


def _prep_body(x_ref, wwt_ref, bw_ref, wkt_ref, bk_ref, kern_ref, idx_ref):
    i = pl.program_id(0)
    b = i // (_L // _T)
    l0 = (i % (_L // _T)) * _T
    xb = x_ref[...]
    wave = jnp.dot(xb, wwt_ref[...], preferred_element_type=jnp.float32) + bw_ref[...]
    wave = wave * jax.nn.sigmoid(wave)
    freq = jax.nn.sigmoid(wave[:, :_H]) * (_MAXF - _MINF) + _MINF
    phase = jnp.tanh(wave[:, _H:]) * _MAXF
    fa = jnp.mean(freq, axis=1, keepdims=True)
    pa = jnp.mean(phase, axis=1, keepdims=True)
    km = jnp.dot(xb, wkt_ref[...], preferred_element_type=jnp.float32) + bk_ref[...]
    km = km * jax.nn.sigmoid(km)
    centers = (l0 + lax.broadcasted_iota(jnp.int32, (_T, 1), 0)).astype(jnp.float32)
    svec = lax.broadcasted_iota(jnp.int32, (_T, _S), 1).astype(jnp.float32) - float(_HALF)
    rel = svec * fa + pa
    pos = (centers + svec * fa) + pa
    validf = ((pos >= 0.0) & (pos < float(_L))).astype(jnp.float32)
    idx_ref[...] = jnp.clip(pos.astype(jnp.int32), 0, _L - 1) + b * _L
    fidx = jnp.clip((rel + _MAXR) / (2.0 * _MAXR), 0.0, 1.0) * float(_K - 1)
    ifl = jnp.clip(fidx.astype(jnp.int32), 0, _K - 2)
    wc = fidx - ifl.astype(jnp.float32)
    wf = 1.0 - wc
    for h in range(_H):
        km_h = km[:, h * _K:(h + 1) * _K]
        kf = jnp.take_along_axis(km_h, ifl, axis=1)
        kc = jnp.take_along_axis(km_h, ifl + 1, axis=1)
        kh = (kf * wf + kc * wc) * validf
        den = jnp.sum(kh, axis=1, keepdims=True) + 1e-8
        kern_ref[:, h * _S:(h + 1) * _S] = kh / den


def _prep_call(x2d, wwt, bw2, wkt, bk2):
    return pl.pallas_call(
        _prep_body,
        grid=(_NBLK,),
        in_specs=[
            pl.BlockSpec((_T, _C), lambda i: (i, 0)),
            pl.BlockSpec((_C, 2 * _H), lambda i: (0, 0)),
            pl.BlockSpec((1, 2 * _H), lambda i: (0, 0)),
            pl.BlockSpec((_C, _H * _K), lambda i: (0, 0)),
            pl.BlockSpec((1, _H * _K), lambda i: (0, 0)),
        ],
        out_specs=[
            pl.BlockSpec((_T, _S * _H), lambda i: (i, 0)),
            pl.BlockSpec((_T, _S), lambda i: (i, 0)),
        ],
        out_shape=[
            jax.ShapeDtypeStruct((_BL, _S * _H), jnp.float32),
            jax.ShapeDtypeStruct((_BL, _S), jnp.int32),
        ],
    )(x2d, wwt, bw2, wkt, bk2)


def _sc_body(x_hbm, idx_hbm, kern_hbm, out_hbm, idx_v, kern_v, vals_v, acc_v, sem):
    cid = lax.axis_index("c")
    sid = lax.axis_index("s")
    wid = sid * 2 + cid
    base = wid * _RPW

    lane = lax.iota(jnp.int32, 16) * 64
    hstep = lax.iota(jnp.int32, 16) * _S

    def row(i, carry):
        r = base + i
        pltpu.sync_copy(idx_hbm.at[r], idx_v)
        cp = pltpu.async_copy(x_hbm.at[idx_v], vals_v, sem)
        pltpu.sync_copy(kern_hbm.at[r], kern_v)
        cp.wait()
        kws = tuple(plsc.load_gather(kern_v, [hstep + s]) for s in range(_S))
        z = jnp.zeros((16,), jnp.float32)

        def dbody(d, c2):
            acc = z
            cidx = d + lane
            for s in range(_S):
                vv = plsc.load_gather(vals_v, [jnp.full((16,), s, jnp.int32), cidx])
                acc = acc + kws[s] * vv
            plsc.store_scatter(acc_v, [cidx], acc)
            return c2

        lax.fori_loop(0, 64, dbody, 0)
        pltpu.sync_copy(acc_v, out_hbm.at[r])
        return carry

    lax.fori_loop(0, _RPW, row, 0)


def _sc_call(x2d, idx2d, kern2d):
    mesh = plsc.VectorSubcoreMesh(core_axis_name="c", subcore_axis_name="s")
    fn = functools.partial(
        pl.kernel,
        out_type=jax.ShapeDtypeStruct((_BL, _C), jnp.float32),
        mesh=mesh,
        scratch_types=[
            pltpu.VMEM((_S,), jnp.int32),
            pltpu.VMEM((_S * _H,), jnp.float32),
            pltpu.VMEM((_S, _C), jnp.float32),
            pltpu.VMEM((_C,), jnp.float32),
            pltpu.SemaphoreType.DMA,
        ],
        compiler_params=pltpu.CompilerParams(
            use_tc_tiling_on_sc=False, needs_layout_passes=False
        ),
    )(_sc_body)
    return fn(x2d, idx2d, kern2d)


def _out_body(h_ref, wot_ref, o_ref):
    acc = jnp.dot(h_ref[...], wot_ref[...], preferred_element_type=jnp.float32)
    o_ref[...] = acc * jax.nn.sigmoid(acc)


def _out_call(hidden2d, wot):
    return pl.pallas_call(
        _out_body,
        grid=(_NBLK,),
        in_specs=[
            pl.BlockSpec((_T, _C), lambda i: (i, 0)),
            pl.BlockSpec((_C, _C), lambda i: (0, 0)),
        ],
        out_specs=pl.BlockSpec((_T, _C), lambda i: (i, 0)),
        out_shape=jax.ShapeDtypeStruct((_BL, _C), jnp.float32),
    )(hidden2d, wot)


@jax.jit
def kernel(x, Ww, bw, Wk, bk, Wo):
    x2d = x.reshape(_BL, _C)
    kern2d, idx2d = _prep_call(x2d, Ww.T, bw.reshape(1, -1), Wk.T, bk.reshape(1, -1))
    hidden2d = _sc_call(x2d, idx2d, kern2d)
    out2d = _out_call(hidden2d, Wo.T)
    return out2d.reshape(_B, _L, _C)


# speedup vs baseline: 1.1092x; 1.1092x over previous
import functools

import jax
import jax.numpy as jnp
from jax import lax
from jax.experimental import pallas as pl
from jax.experimental.pallas import tpu as pltpu
from jax.experimental.pallas import tpu_sc as plsc

_B, _L, _C = 2, 2048, 1024
_H, _K = 16, 64
_S = 33
_HALF = 16
_MAXF, _MINF = 16.0, 1.0
_MAXR = _HALF * _MAXF
_BL = _B * _L
_T = 256
_NBLK = _BL // _T

_NW = 32
_RPW = _BL // _NW


def _prep_body(x_ref, wwt_ref, bw_ref, wkt_ref, bk_ref, kern_ref, idx_ref):
    i = pl.program_id(0)
    b = i // (_L // _T)
    l0 = (i % (_L // _T)) * _T
    xb = x_ref[...]
    wave = jnp.dot(xb, wwt_ref[...], preferred_element_type=jnp.float32) + bw_ref[...]
    wave = wave * jax.nn.sigmoid(wave)
    freq = jax.nn.sigmoid(wave[:, :_H]) * (_MAXF - _MINF) + _MINF
    phase = jnp.tanh(wave[:, _H:]) * _MAXF
    fa = jnp.mean(freq, axis=1, keepdims=True)
    pa = jnp.mean(phase, axis=1, keepdims=True)
    km = jnp.dot(xb, wkt_ref[...], preferred_element_type=jnp.float32) + bk_ref[...]
    km = km * jax.nn.sigmoid(km)
    centers = (l0 + lax.broadcasted_iota(jnp.int32, (_T, 1), 0)).astype(jnp.float32)
    svec = lax.broadcasted_iota(jnp.int32, (_T, _S), 1).astype(jnp.float32) - float(_HALF)
    rel = svec * fa + pa
    pos = (centers + svec * fa) + pa
    validf = ((pos >= 0.0) & (pos < float(_L))).astype(jnp.float32)
    idx_ref[...] = jnp.clip(pos.astype(jnp.int32), 0, _L - 1) + b * _L
    fidx = jnp.clip((rel + _MAXR) / (2.0 * _MAXR), 0.0, 1.0) * float(_K - 1)
    ifl = jnp.clip(fidx.astype(jnp.int32), 0, _K - 2)
    wc = fidx - ifl.astype(jnp.float32)
    wf = 1.0 - wc
    for h in range(_H):
        km_h = km[:, h * _K:(h + 1) * _K]
        kf = jnp.take_along_axis(km_h, ifl, axis=1)
        kc = jnp.take_along_axis(km_h, ifl + 1, axis=1)
        kh = (kf * wf + kc * wc) * validf
        den = jnp.sum(kh, axis=1, keepdims=True) + 1e-8
        kern_ref[:, h * _S:(h + 1) * _S] = kh / den


def _prep_call(x2d, wwt, bw2, wkt, bk2):
    return pl.pallas_call(
        _prep_body,
        grid=(_NBLK,),
        in_specs=[
            pl.BlockSpec((_T, _C), lambda i: (i, 0)),
            pl.BlockSpec((_C, 2 * _H), lambda i: (0, 0)),
            pl.BlockSpec((1, 2 * _H), lambda i: (0, 0)),
            pl.BlockSpec((_C, _H * _K), lambda i: (0, 0)),
            pl.BlockSpec((1, _H * _K), lambda i: (0, 0)),
        ],
        out_specs=[
            pl.BlockSpec((_T, _S * _H), lambda i: (i, 0)),
            pl.BlockSpec((_T, _S), lambda i: (i, 0)),
        ],
        out_shape=[
            jax.ShapeDtypeStruct((_BL, _S * _H), jnp.float32),
            jax.ShapeDtypeStruct((_BL, _S), jnp.int32),
        ],
    )(x2d, wwt, bw2, wkt, bk2)


_CH = 16


def _sc_body(x_hbm, idx_hbm, kern_hbm, out_hbm, idx_ch, kern_ch, vals0, vals1,
             out_ch, sem0, sem1):
    cid = lax.axis_index("c")
    sid = lax.axis_index("s")
    wid = sid * 2 + cid
    base = wid * _RPW

    lane = lax.iota(jnp.int32, 16) * 64
    hstep = lax.iota(jnp.int32, 16) * _S
    z = jnp.zeros((16,), jnp.float32)
    bufs = (vals0, vals1)
    sems = (sem0, sem1)

    def chunk(ci, carry):
        c0 = base + ci * _CH
        pltpu.sync_copy(idx_hbm.at[pl.ds(c0, _CH)], idx_ch)
        pltpu.sync_copy(kern_hbm.at[pl.ds(c0, _CH)], kern_ch)
        pltpu.async_copy(x_hbm.at[idx_ch.at[0]], bufs[0], sems[0])
        pltpu.async_copy(x_hbm.at[idx_ch.at[1]], bufs[1], sems[1])

        def pair(p, c2):
            for q in range(2):
                j = 2 * p + q
                cur = bufs[q]
                pltpu.make_async_copy(
                    x_hbm.at[pl.ds(0, _S)], cur, sems[q]).wait()
                jf = jnp.full((16,), 1, jnp.int32) * j
                kws = tuple(
                    plsc.load_gather(kern_ch, [jf, hstep + s]) for s in range(_S))

                def dbody(dq, c3, cur=cur, kws=kws, jf=jf):
                    for k in range(4):
                        cidx = dq * 4 + k + lane
                        acc = z
                        for s in range(_S):
                            vv = plsc.load_gather(
                                cur, [jnp.full((16,), s, jnp.int32), cidx])
                            acc = acc + kws[s] * vv
                        plsc.store_scatter(out_ch, [jf, cidx], acc)
                    return c3

                lax.fori_loop(0, 16, dbody, 0)

                @pl.when(j + 2 < _CH)
                def _():
                    pltpu.async_copy(x_hbm.at[idx_ch.at[j + 2]], cur, sems[q])
            return c2

        lax.fori_loop(0, _CH // 2, pair, 0)
        pltpu.sync_copy(out_ch, out_hbm.at[pl.ds(c0, _CH)])
        return carry

    lax.fori_loop(0, _RPW // _CH, chunk, 0)


def _sc_call(x2d, idx2d, kern2d):
    mesh = plsc.VectorSubcoreMesh(core_axis_name="c", subcore_axis_name="s")
    fn = functools.partial(
        pl.kernel,
        out_type=jax.ShapeDtypeStruct((_BL, _C), jnp.float32),
        mesh=mesh,
        scratch_types=[
            pltpu.VMEM((_CH, _S), jnp.int32),
            pltpu.VMEM((_CH, _S * _H), jnp.float32),
            pltpu.VMEM((_S, _C), jnp.float32),
            pltpu.VMEM((_S, _C), jnp.float32),
            pltpu.VMEM((_CH, _C), jnp.float32),
            pltpu.SemaphoreType.DMA,
            pltpu.SemaphoreType.DMA,
        ],
        compiler_params=pltpu.CompilerParams(
            use_tc_tiling_on_sc=False, needs_layout_passes=False
        ),
    )(_sc_body)
    return fn(x2d, idx2d, kern2d)


def _out_body(h_ref, wot_ref, o_ref):
    acc = jnp.dot(h_ref[...], wot_ref[...], preferred_element_type=jnp.float32)
    o_ref[...] = acc * jax.nn.sigmoid(acc)


def _out_call(hidden2d, wot):
    return pl.pallas_call(
        _out_body,
        grid=(_NBLK,),
        in_specs=[
            pl.BlockSpec((_T, _C), lambda i: (i, 0)),
            pl.BlockSpec((_C, _C), lambda i: (0, 0)),
        ],
        out_specs=pl.BlockSpec((_T, _C), lambda i: (i, 0)),
        out_shape=jax.ShapeDtypeStruct((_BL, _C), jnp.float32),
    )(hidden2d, wot)


@jax.jit
def kernel(x, Ww, bw, Wk, bk, Wo):
    x2d = x.reshape(_BL, _C)
    kern2d, idx2d = _prep_call(x2d, Ww.T, bw.reshape(1, -1), Wk.T, bk.reshape(1, -1))
    hidden2d = _sc_call(x2d, idx2d, kern2d)
    out2d = _out_call(hidden2d, Wo.T)
    return out2d.reshape(_B, _L, _C)


# speedup vs baseline: 3.5767x; 3.2244x over previous
import functools

import jax
import jax.numpy as jnp
from jax import lax
from jax.experimental import pallas as pl
from jax.experimental.pallas import tpu as pltpu
from jax.experimental.pallas import tpu_sc as plsc

_B, _L, _C = 2, 2048, 1024
_H, _K = 16, 64
_S = 33
_HALF = 16
_MAXF, _MINF = 16.0, 1.0
_MAXR = _HALF * _MAXF
_BL = _B * _L
_T = 256
_NBLK = _BL // _T

_NW = 32
_RPW = _BL // _NW


def _prep_body(x_ref, wwt_ref, bw_ref, wkt_ref, bk_ref, kern_ref, idx_ref):
    i = pl.program_id(0)
    b = i // (_L // _T)
    l0 = (i % (_L // _T)) * _T
    xb = x_ref[...]
    wave = jnp.dot(xb, wwt_ref[...], preferred_element_type=jnp.float32) + bw_ref[...]
    wave = wave * jax.nn.sigmoid(wave)
    freq = jax.nn.sigmoid(wave[:, :_H]) * (_MAXF - _MINF) + _MINF
    phase = jnp.tanh(wave[:, _H:]) * _MAXF
    fa = jnp.mean(freq, axis=1, keepdims=True)
    pa = jnp.mean(phase, axis=1, keepdims=True)
    km = jnp.dot(xb, wkt_ref[...], preferred_element_type=jnp.float32) + bk_ref[...]
    km = km * jax.nn.sigmoid(km)
    centers = (l0 + lax.broadcasted_iota(jnp.int32, (_T, 1), 0)).astype(jnp.float32)
    svec = lax.broadcasted_iota(jnp.int32, (_T, _S), 1).astype(jnp.float32) - float(_HALF)
    rel = svec * fa + pa
    pos = (centers + svec * fa) + pa
    validf = ((pos >= 0.0) & (pos < float(_L))).astype(jnp.float32)
    idx_ref[...] = jnp.clip(pos.astype(jnp.int32), 0, _L - 1) + b * _L
    fidx = jnp.clip((rel + _MAXR) / (2.0 * _MAXR), 0.0, 1.0) * float(_K - 1)
    ifl = jnp.clip(fidx.astype(jnp.int32), 0, _K - 2)
    wc = fidx - ifl.astype(jnp.float32)
    wf = 1.0 - wc
    repidx = lax.broadcasted_iota(jnp.int32, (_T, _S * 16), 1) // 16
    for h in range(_H):
        km_h = km[:, h * _K:(h + 1) * _K]
        kf = jnp.take_along_axis(km_h, ifl, axis=1)
        kc = jnp.take_along_axis(km_h, ifl + 1, axis=1)
        kh = (kf * wf + kc * wc) * validf
        den = jnp.sum(kh, axis=1, keepdims=True) + 1e-8
        khn = kh / den
        kern_ref[:, h * _S * 16:(h + 1) * _S * 16] = jnp.take_along_axis(
            khn, repidx, axis=1)


def _prep_call(x2d, wwt, bw2, wkt, bk2):
    return pl.pallas_call(
        _prep_body,
        grid=(_NBLK,),
        in_specs=[
            pl.BlockSpec((_T, _C), lambda i: (i, 0)),
            pl.BlockSpec((_C, 2 * _H), lambda i: (0, 0)),
            pl.BlockSpec((1, 2 * _H), lambda i: (0, 0)),
            pl.BlockSpec((_C, _H * _K), lambda i: (0, 0)),
            pl.BlockSpec((1, _H * _K), lambda i: (0, 0)),
        ],
        out_specs=[
            pl.BlockSpec((_T, _S * _H * 16), lambda i: (i, 0)),
            pl.BlockSpec((_T, _S), lambda i: (i, 0)),
        ],
        out_shape=[
            jax.ShapeDtypeStruct((_BL, _S * _H * 16), jnp.float32),
            jax.ShapeDtypeStruct((_BL, _S), jnp.int32),
        ],
    )(x2d, wwt, bw2, wkt, bk2)


_CH = 16
_WROW = _S * _H * 16


def _sc_body(x_hbm, idx_hbm, wexp_hbm, out_hbm, idx_ch, wexp0, wexp1, vals0,
             vals1, out_ch, semv0, semv1, semw0, semw1):
    cid = lax.axis_index("c")
    sid = lax.axis_index("s")
    wid = sid * 2 + cid
    base = wid * _RPW

    z = jnp.zeros((16,), jnp.float32)
    vbufs = (vals0, vals1)
    wbufs = (wexp0, wexp1)
    vsems = (semv0, semv1)
    wsems = (semw0, semw1)

    def chunk(ci, carry):
        c0 = base + ci * _CH
        pltpu.sync_copy(idx_hbm.at[pl.ds(c0, _CH)], idx_ch)
        for q in range(2):
            pltpu.async_copy(x_hbm.at[idx_ch.at[q]], vbufs[q], vsems[q])
            pltpu.async_copy(wexp_hbm.at[c0 + q], wbufs[q], wsems[q])

        def pair(p, c2):
            for q in range(2):
                j = 2 * p + q
                vcur = vbufs[q]
                wcur = wbufs[q]
                pltpu.make_async_copy(
                    x_hbm.at[pl.ds(0, _S)], vcur, vsems[q]).wait()
                pltpu.make_async_copy(wexp_hbm.at[0], wcur, wsems[q]).wait()
                orow = out_ch.at[j]

                def hbody(h, c3, vcur=vcur, wcur=wcur, orow=orow):
                    wb = h * (_S * 16)
                    cb = h * 64
                    a0 = z
                    a1 = z
                    a2 = z
                    a3 = z
                    for s in range(_S):
                        w = wcur[pl.ds(wb + s * 16, 16)]
                        vrow = vcur.at[s]
                        a0 = a0 + w * vrow[pl.ds(cb, 16)]
                        a1 = a1 + w * vrow[pl.ds(cb + 16, 16)]
                        a2 = a2 + w * vrow[pl.ds(cb + 32, 16)]
                        a3 = a3 + w * vrow[pl.ds(cb + 48, 16)]
                    orow[pl.ds(cb, 16)] = a0
                    orow[pl.ds(cb + 16, 16)] = a1
                    orow[pl.ds(cb + 32, 16)] = a2
                    orow[pl.ds(cb + 48, 16)] = a3
                    return c3

                lax.fori_loop(0, _H, hbody, 0)

                @pl.when(j + 2 < _CH)
                def _():
                    pltpu.async_copy(x_hbm.at[idx_ch.at[j + 2]], vcur, vsems[q])
                    pltpu.async_copy(wexp_hbm.at[c0 + j + 2], wcur, wsems[q])
            return c2

        lax.fori_loop(0, _CH // 2, pair, 0)
        pltpu.sync_copy(out_ch, out_hbm.at[pl.ds(c0, _CH)])
        return carry

    lax.fori_loop(0, _RPW // _CH, chunk, 0)


def _sc_call(x2d, idx2d, wexp2d):
    mesh = plsc.VectorSubcoreMesh(core_axis_name="c", subcore_axis_name="s")
    fn = functools.partial(
        pl.kernel,
        out_type=jax.ShapeDtypeStruct((_BL, _C), jnp.float32),
        mesh=mesh,
        scratch_types=[
            pltpu.VMEM((_CH, _S), jnp.int32),
            pltpu.VMEM((_WROW,), jnp.float32),
            pltpu.VMEM((_WROW,), jnp.float32),
            pltpu.VMEM((_S, _C), jnp.float32),
            pltpu.VMEM((_S, _C), jnp.float32),
            pltpu.VMEM((_CH, _C), jnp.float32),
            pltpu.SemaphoreType.DMA,
            pltpu.SemaphoreType.DMA,
            pltpu.SemaphoreType.DMA,
            pltpu.SemaphoreType.DMA,
        ],
        compiler_params=pltpu.CompilerParams(
            use_tc_tiling_on_sc=False, needs_layout_passes=False
        ),
    )(_sc_body)
    return fn(x2d, idx2d, wexp2d)


def _out_body(h_ref, wot_ref, o_ref):
    acc = jnp.dot(h_ref[...], wot_ref[...], preferred_element_type=jnp.float32)
    o_ref[...] = acc * jax.nn.sigmoid(acc)


def _out_call(hidden2d, wot):
    return pl.pallas_call(
        _out_body,
        grid=(_NBLK,),
        in_specs=[
            pl.BlockSpec((_T, _C), lambda i: (i, 0)),
            pl.BlockSpec((_C, _C), lambda i: (0, 0)),
        ],
        out_specs=pl.BlockSpec((_T, _C), lambda i: (i, 0)),
        out_shape=jax.ShapeDtypeStruct((_BL, _C), jnp.float32),
    )(hidden2d, wot)


@jax.jit
def kernel(x, Ww, bw, Wk, bk, Wo):
    x2d = x.reshape(_BL, _C)
    kern2d, idx2d = _prep_call(x2d, Ww.T, bw.reshape(1, -1), Wk.T, bk.reshape(1, -1))
    hidden2d = _sc_call(x2d, idx2d, kern2d)
    out2d = _out_call(hidden2d, Wo.T)
    return out2d.reshape(_B, _L, _C)


# speedup vs baseline: 4.0791x; 1.1405x over previous
import functools

import numpy as np

import jax
import jax.numpy as jnp
from jax import lax
from jax.experimental import pallas as pl
from jax.experimental.pallas import tpu as pltpu
from jax.experimental.pallas import tpu_sc as plsc

_B, _L, _C = 2, 2048, 1024
_H, _K = 16, 64
_S = 33
_HALF = 16
_MAXF, _MINF = 16.0, 1.0
_MAXR = _HALF * _MAXF
_BL = _B * _L
_T = 256
_NBLK = _BL // _T

_NW = 32
_RPW = _BL // _NW


def _prep_body(x_ref, wwt_ref, bw_ref, wkt_ref, bk_ref, kern_ref, idx_ref):
    i = pl.program_id(0)
    b = i // (_L // _T)
    l0 = (i % (_L // _T)) * _T
    xb = x_ref[...]
    wave = jnp.dot(xb, wwt_ref[...], preferred_element_type=jnp.float32) + bw_ref[...]
    wave = wave * jax.nn.sigmoid(wave)
    freq = jax.nn.sigmoid(wave[:, :_H]) * (_MAXF - _MINF) + _MINF
    phase = jnp.tanh(wave[:, _H:]) * _MAXF
    fa = jnp.mean(freq, axis=1, keepdims=True)
    pa = jnp.mean(phase, axis=1, keepdims=True)
    km = jnp.dot(xb, wkt_ref[...], preferred_element_type=jnp.float32) + bk_ref[...]
    km = km * jax.nn.sigmoid(km)
    centers = (l0 + lax.broadcasted_iota(jnp.int32, (_T, 1), 0)).astype(jnp.float32)
    svec = lax.broadcasted_iota(jnp.int32, (_T, _S), 1).astype(jnp.float32) - float(_HALF)
    rel = svec * fa + pa
    pos = (centers + svec * fa) + pa
    validf = ((pos >= 0.0) & (pos < float(_L))).astype(jnp.float32)
    idx_ref[...] = jnp.clip(pos.astype(jnp.int32), 0, _L - 1) + b * _L
    fidx = jnp.clip((rel + _MAXR) / (2.0 * _MAXR), 0.0, 1.0) * float(_K - 1)
    ifl = jnp.clip(fidx.astype(jnp.int32), 0, _K - 2)
    wc = fidx - ifl.astype(jnp.float32)
    wf = 1.0 - wc
    repidx = lax.broadcasted_iota(jnp.int32, (_T, _S * 16), 1) // 16
    for h in range(_H):
        km_h = km[:, h * _K:(h + 1) * _K]
        kf = jnp.take_along_axis(km_h, ifl, axis=1)
        kc = jnp.take_along_axis(km_h, ifl + 1, axis=1)
        kh = (kf * wf + kc * wc) * validf
        den = jnp.sum(kh, axis=1, keepdims=True) + 1e-8
        khn = kh / den
        kern_ref[:, h * _S * 16:(h + 1) * _S * 16] = jnp.take_along_axis(
            khn, repidx, axis=1)


def _prep_call(x2d, wwt, bw2, wkt, bk2):
    return pl.pallas_call(
        _prep_body,
        grid=(_NBLK,),
        in_specs=[
            pl.BlockSpec((_T, _C), lambda i: (i, 0)),
            pl.BlockSpec((_C, 2 * _H), lambda i: (0, 0)),
            pl.BlockSpec((1, 2 * _H), lambda i: (0, 0)),
            pl.BlockSpec((_C, _H * _K), lambda i: (0, 0)),
            pl.BlockSpec((1, _H * _K), lambda i: (0, 0)),
        ],
        out_specs=[
            pl.BlockSpec((_T, _S * _H * 16), lambda i: (i, 0)),
            pl.BlockSpec((_T, _S), lambda i: (i, 0)),
        ],
        out_shape=[
            jax.ShapeDtypeStruct((_BL, _S * _H * 16), jnp.float32),
            jax.ShapeDtypeStruct((_BL, _S), jnp.int32),
        ],
    )(x2d, wwt, bw2, wkt, bk2)


_CH = 16
_WROW = _S * _H * 16


def _sc_body(x_hbm, idx_hbm, wexp_hbm, out_hbm, idx_ch, wexp0, wexp1, vals0,
             vals1, out_ch, semv0, semv1, semw0, semw1):
    cid = lax.axis_index("c")
    sid = lax.axis_index("s")
    wid = sid * 2 + cid
    base = wid * _RPW

    z = jnp.zeros((16,), jnp.float32)
    vbufs = (vals0, vals1)
    wbufs = (wexp0, wexp1)
    vsems = (semv0, semv1)
    wsems = (semw0, semw1)

    def chunk(ci, carry):
        c0 = base + ci * _CH
        pltpu.sync_copy(idx_hbm.at[pl.ds(c0, _CH)], idx_ch)
        for q in range(2):
            pltpu.async_copy(x_hbm.at[idx_ch.at[q]], vbufs[q], vsems[q])
            pltpu.async_copy(wexp_hbm.at[c0 + q], wbufs[q], wsems[q])

        def pair(p, c2):
            for q in range(2):
                j = 2 * p + q
                vcur = vbufs[q]
                wcur = wbufs[q]
                pltpu.make_async_copy(
                    x_hbm.at[pl.ds(0, _S)], vcur, vsems[q]).wait()
                pltpu.make_async_copy(wexp_hbm.at[0], wcur, wsems[q]).wait()
                orow = out_ch.at[j]

                def hbody(h, c3, vcur=vcur, wcur=wcur, orow=orow):
                    wb = h * (_S * 16)
                    cb = h * 64
                    a0 = z
                    a1 = z
                    a2 = z
                    a3 = z
                    for s in range(_S):
                        w = wcur[pl.ds(wb + s * 16, 16)]
                        vrow = vcur.at[s]
                        e0, o0 = plsc.unpack(
                            vrow[pl.ds(cb, 32)],
                            format=plsc.PackFormat.INTERLEAVED,
                            preferred_element_type=jnp.float32)
                        e1, o1 = plsc.unpack(
                            vrow[pl.ds(cb + 32, 32)],
                            format=plsc.PackFormat.INTERLEAVED,
                            preferred_element_type=jnp.float32)
                        a0 = a0 + w * e0
                        a1 = a1 + w * o0
                        a2 = a2 + w * e1
                        a3 = a3 + w * o1
                    orow[pl.ds(cb, 16)] = a0
                    orow[pl.ds(cb + 16, 16)] = a1
                    orow[pl.ds(cb + 32, 16)] = a2
                    orow[pl.ds(cb + 48, 16)] = a3
                    return c3

                lax.fori_loop(0, _H, hbody, 0)

                @pl.when(j + 2 < _CH)
                def _():
                    pltpu.async_copy(x_hbm.at[idx_ch.at[j + 2]], vcur, vsems[q])
                    pltpu.async_copy(wexp_hbm.at[c0 + j + 2], wcur, wsems[q])
            return c2

        lax.fori_loop(0, _CH // 2, pair, 0)
        pltpu.sync_copy(out_ch, out_hbm.at[pl.ds(c0, _CH)])
        return carry

    lax.fori_loop(0, _RPW // _CH, chunk, 0)


def _sc_call(x2d, idx2d, wexp2d):
    mesh = plsc.VectorSubcoreMesh(core_axis_name="c", subcore_axis_name="s")
    fn = functools.partial(
        pl.kernel,
        out_type=jax.ShapeDtypeStruct((_BL, _C), jnp.float32),
        mesh=mesh,
        scratch_types=[
            pltpu.VMEM((_CH, _S), jnp.int32),
            pltpu.VMEM((_WROW,), jnp.float32),
            pltpu.VMEM((_WROW,), jnp.float32),
            pltpu.VMEM((_S, _C), jnp.bfloat16),
            pltpu.VMEM((_S, _C), jnp.bfloat16),
            pltpu.VMEM((_CH, _C), jnp.float32),
            pltpu.SemaphoreType.DMA,
            pltpu.SemaphoreType.DMA,
            pltpu.SemaphoreType.DMA,
            pltpu.SemaphoreType.DMA,
        ],
        compiler_params=pltpu.CompilerParams(
            use_tc_tiling_on_sc=False, needs_layout_passes=False
        ),
    )(_sc_body)
    return fn(x2d, idx2d, wexp2d)


def _out_body(h_ref, wot_ref, o_ref):
    acc = jnp.dot(h_ref[...], wot_ref[...], preferred_element_type=jnp.float32)
    o_ref[...] = acc * jax.nn.sigmoid(acc)


def _out_call(hidden2d, wot):
    return pl.pallas_call(
        _out_body,
        grid=(_NBLK,),
        in_specs=[
            pl.BlockSpec((_T, _C), lambda i: (i, 0)),
            pl.BlockSpec((_C, _C), lambda i: (0, 0)),
        ],
        out_specs=pl.BlockSpec((_T, _C), lambda i: (i, 0)),
        out_shape=jax.ShapeDtypeStruct((_BL, _C), jnp.float32),
    )(hidden2d, wot)


_PERM = np.concatenate([
    np.concatenate([g * 32 + np.arange(0, 32, 2), g * 32 + np.arange(1, 32, 2)])
    for g in range(_C // 32)
])


@jax.jit
def kernel(x, Ww, bw, Wk, bk, Wo):
    x2d = x.reshape(_BL, _C)
    xbf = x2d.astype(jnp.bfloat16)
    wexp2d, idx2d = _prep_call(x2d, Ww.T, bw.reshape(1, -1), Wk.T, bk.reshape(1, -1))
    hidden2d = _sc_call(xbf, idx2d, wexp2d)
    out2d = _out_call(hidden2d, Wo.T[_PERM])
    return out2d.reshape(_B, _L, _C)
